# baseline (speedup 1.0000x reference)
import functools

import jax
import jax.numpy as jnp
from jax import lax
from jax.experimental import pallas as pl
from jax.experimental.pallas import tpu as pltpu
from jax.experimental.pallas import tpu_sc as plsc

VOCAB = 100000
D = 16
B = 16384

_info = plsc.get_sparse_core_info()
NC, NS, L = _info.num_cores, _info.num_subcores, _info.num_lanes
HALF = B // NC
CHUNK = 4096
NCH = HALF // CHUNK
RED = HALF // NS

_mesh = plsc.VectorSubcoreMesh(core_axis_name="c", subcore_axis_name="s")


@functools.partial(
    pl.kernel,
    out_type=jax.ShapeDtypeStruct((B,), jnp.float32),
    mesh=_mesh,
    scratch_types=[
        pltpu.VMEM_SHARED((NS, HALF), jnp.float32),
        pltpu.VMEM((VOCAB,), jnp.float32),
        pltpu.VMEM((CHUNK,), jnp.int32),
        pltpu.VMEM((CHUNK,), jnp.int32),
        pltpu.VMEM((CHUNK,), jnp.float32),
        pltpu.VMEM((NS, RED), jnp.float32),
        pltpu.VMEM((RED,), jnp.float32),
        pltpu.SemaphoreType.DMA,
    ],
    compiler_params=pltpu.CompilerParams(
        needs_layout_passes=False, use_tc_tiling_on_sc=True),
)
def _w2v_kernel(tableT_hbm, pairT_hbm, out_hbm,
                shared, slice_v, cen_ch, ctx_ch, prod_ch, red_buf, acc,
                sem):
    d = lax.axis_index("s")
    c = lax.axis_index("c")
    hbase = c * HALF

    slice_cp = pltpu.async_copy(tableT_hbm.at[d], slice_v, sem)
    pltpu.sync_copy(pairT_hbm.at[0, pl.ds(hbase, CHUNK)], cen_ch)
    pltpu.sync_copy(pairT_hbm.at[1, pl.ds(hbase, CHUNK)], ctx_ch)
    slice_cp.wait()

    def chunk_body(ch, carry):
        base = hbase + ch * CHUNK

        @pl.when(ch > 0)
        def _():
            pltpu.sync_copy(pairT_hbm.at[0, pl.ds(base, CHUNK)], cen_ch)
            pltpu.sync_copy(pairT_hbm.at[1, pl.ds(base, CHUNK)], ctx_ch)

        @plsc.parallel_loop(0, CHUNK, step=L, unroll=8)
        def group_body(o):
            ci = cen_ch[pl.ds(o, L)]
            xi = ctx_ch[pl.ds(o, L)]
            cv = plsc.load_gather(slice_v, [ci])
            xv = plsc.load_gather(slice_v, [xi])
            prod_ch[pl.ds(o, L)] = cv * xv
        pltpu.sync_copy(prod_ch, shared.at[d, pl.ds(ch * CHUNK, CHUNK)])
        return carry

    lax.fori_loop(0, NCH, chunk_body, None)
    plsc.subcore_barrier()

    pltpu.sync_copy(shared.at[:, pl.ds(d * RED, RED)], red_buf)

    def red_body(g, carry):
        def dd_body(dd, v):
            return v + red_buf[dd, pl.ds(g * L, L)]

        acc[pl.ds(g * L, L)] = lax.fori_loop(
            0, NS, dd_body, jnp.zeros((L,), jnp.float32))
        return carry

    lax.fori_loop(0, RED // L, red_body, None)
    pltpu.sync_copy(acc, out_hbm.at[pl.ds(hbase + d * RED, RED)])


def kernel(pair, label, table):
    del label
    pair = jnp.reshape(pair, (-1, 2)).astype(jnp.int32)
    sims = _w2v_kernel(table.T, pair.T)
    return jnp.reshape(sims, (B, 1))


# speedup vs baseline: 1.0434x; 1.0434x over previous
import functools

import jax
import jax.numpy as jnp
from jax import lax
from jax.experimental import pallas as pl
from jax.experimental.pallas import tpu as pltpu
from jax.experimental.pallas import tpu_sc as plsc

VOCAB = 100000
D = 16
B = 16384

_info = plsc.get_sparse_core_info()
NC, NS, L = _info.num_cores, _info.num_subcores, _info.num_lanes
HALF = B // NC
CHUNK = 4096
NCH = HALF // CHUNK
RED = HALF // NS

_mesh = plsc.VectorSubcoreMesh(core_axis_name="c", subcore_axis_name="s")


@functools.partial(
    pl.kernel,
    out_type=jax.ShapeDtypeStruct((B,), jnp.float32),
    mesh=_mesh,
    scratch_types=[
        pltpu.VMEM_SHARED((NS, HALF), jnp.float32),
        pltpu.VMEM((VOCAB,), jnp.float32),
        pltpu.VMEM((CHUNK,), jnp.int32),
        pltpu.VMEM((CHUNK,), jnp.int32),
        pltpu.VMEM((CHUNK,), jnp.float32),
        pltpu.VMEM((NS, RED), jnp.float32),
        pltpu.VMEM((RED,), jnp.float32),
        pltpu.SemaphoreType.DMA,
    ],
    compiler_params=pltpu.CompilerParams(
        needs_layout_passes=False, use_tc_tiling_on_sc=True),
)
def _w2v_kernel(tableT_hbm, pairT_hbm, out_hbm,
                shared, slice_v, cen_ch, ctx_ch, prod_ch, red_buf, acc,
                sem):
    d = lax.axis_index("s")
    c = lax.axis_index("c")
    hbase = c * HALF

    slice_cp = pltpu.async_copy(tableT_hbm.at[d], slice_v, sem)
    pltpu.sync_copy(pairT_hbm.at[0, pl.ds(hbase, CHUNK)], cen_ch)
    pltpu.sync_copy(pairT_hbm.at[1, pl.ds(hbase, CHUNK)], ctx_ch)
    slice_cp.wait()

    def chunk_body(ch, carry):
        base = hbase + ch * CHUNK

        @pl.when(ch > 0)
        def _():
            pltpu.sync_copy(pairT_hbm.at[0, pl.ds(base, CHUNK)], cen_ch)
            pltpu.sync_copy(pairT_hbm.at[1, pl.ds(base, CHUNK)], ctx_ch)

        @plsc.parallel_loop(0, CHUNK, step=L, unroll=8)
        def group_body(o):
            ci = cen_ch[pl.ds(o, L)]
            xi = ctx_ch[pl.ds(o, L)]
            cv = plsc.load_gather(slice_v, [ci])
            xv = plsc.load_gather(slice_v, [xi])
            prod_ch[pl.ds(o, L)] = cv * xv
        pltpu.sync_copy(prod_ch, shared.at[d, pl.ds(ch * CHUNK, CHUNK)])
        return carry

    lax.fori_loop(0, NCH, chunk_body, None)
    plsc.subcore_barrier()

    pltpu.sync_copy(shared.at[:, pl.ds(d * RED, RED)], red_buf)

    def red_body(g, carry):
        s = jnp.zeros((L,), jnp.float32)
        for dd in range(NS):
            s = s + red_buf[dd, pl.ds(g * L, L)]
        acc[pl.ds(g * L, L)] = s
        return carry

    lax.fori_loop(0, RED // L, red_body, None)
    pltpu.sync_copy(acc, out_hbm.at[pl.ds(hbase + d * RED, RED)])


def kernel(pair, label, table):
    del label
    pair = jnp.reshape(pair, (-1, 2)).astype(jnp.int32)
    sims = _w2v_kernel(table.T, pair.T)
    return jnp.reshape(sims, (B, 1))
